# baseline (speedup 1.0000x reference)
import functools

import jax
import jax.numpy as jnp
import numpy as np
from jax import lax
from jax.experimental import pallas as pl
from jax.experimental.pallas import tpu as pltpu
from jax.experimental.pallas import tpu_sc as plsc

EPS = 1e-6

NUM_CORES = 2
NUM_SUBCORES = 16
NUM_WORKERS = NUM_CORES * NUM_SUBCORES

B_TOTAL = 4096 * 200
D = 32
CHUNK = 3200


V = 1000000
BQ = 1024
NQ = 245
V4P = NQ * BQ
VP = 4 * V4P
_LAST_CBLK = (V - 1) // BQ


def _xp_body(a_ref, b_ref, c_ref, d_ref, out_ref):
    out_ref[:, 0:32] = a_ref[...].T
    out_ref[:, 32:64] = b_ref[...].T
    out_ref[:, 64:96] = c_ref[...].T
    out_ref[:, 96:128] = d_ref[...].T


def _pack_table(W):
    WT = W.T
    return pl.pallas_call(
        _xp_body,
        grid=(NQ,),
        in_specs=[
            pl.BlockSpec(
                (32, BQ),
                lambda i, r=r: (0, jnp.minimum(r * NQ + i, _LAST_CBLK)))
            for r in range(4)
        ],
        out_specs=pl.BlockSpec((BQ, 128), lambda i: (i, 0)),
        out_shape=jax.ShapeDtypeStruct((V4P, 128), jnp.float32),
    )(WT, WT, WT, WT)


def _sc_gather(idx, table, nb):
    b_per_w = nb // NUM_WORKERS
    n_chunks = b_per_w // CHUNK
    assert b_per_w % CHUNK == 0
    mesh = plsc.VectorSubcoreMesh(core_axis_name="c", subcore_axis_name="s")

    @functools.partial(
        pl.kernel,
        mesh=mesh,
        out_type=jax.ShapeDtypeStruct((nb, D), jnp.float32),
        compiler_params=pltpu.CompilerParams(use_tc_tiling_on_sc=False),
        scratch_types=[
            pltpu.VMEM((CHUNK,), jnp.int32),
            pltpu.VMEM((CHUNK, D), jnp.float32),
            pltpu.SemaphoreType.DMA,
        ],
    )
    def gather_kernel(idx_hbm, table_hbm, out_hbm, idx_v, rows_v, sem):
        wid = lax.axis_index("s") * NUM_CORES + lax.axis_index("c")
        base = wid * b_per_w

        def body(i, carry):
            off = base + i * CHUNK
            pltpu.sync_copy(idx_hbm.at[pl.ds(off, CHUNK)], idx_v)
            pltpu.async_copy(table_hbm.at[idx_v], rows_v, sem).wait()
            pltpu.sync_copy(rows_v, out_hbm.at[pl.ds(off, CHUNK)])
            return carry

        lax.fori_loop(0, n_chunks, body, 0)

    return gather_kernel(idx, table)


PACK = 128 // D
N_DENSE = B_TOTAL // PACK
BLK = 6400


def _tc_body(rows_ref, ones_ref, out_ref):
    x = rows_ref[...]
    s = jax.lax.dot_general(
        x * x, ones_ref[...], (((1,), (0,)), ((), ())),
        precision=jax.lax.Precision.HIGHEST,
        preferred_element_type=jnp.float32,
    )
    x0 = jnp.sqrt(1.0 + s)
    z = jnp.maximum(x0, 1.0 + EPS)
    m = jnp.maximum(x0 * x0 - 1.0, EPS)
    r = jax.lax.rsqrt(m)
    alpha = jnp.log(z + m * r)
    out_ref[...] = (alpha * r) * x


def _tc_project(rows_dense, nd):
    ones_bd = jnp.asarray(
        np.kron(np.eye(PACK, dtype=np.float32), np.ones((D, D), np.float32)))
    grid = (nd // BLK,)
    return pl.pallas_call(
        _tc_body,
        grid=grid,
        in_specs=[
            pl.BlockSpec((BLK, 128), lambda i: (i, 0)),
            pl.BlockSpec((128, 128), lambda i: (0, 0)),
        ],
        out_specs=pl.BlockSpec((BLK, 128), lambda i: (i, 0)),
        out_shape=jax.ShapeDtypeStruct((nd, 128), jnp.float32),
    )(rows_dense, ones_bd)


N_SPLIT = 2
S_PER = 200 // N_SPLIT
NB_H = 4096 * S_PER

XB = 1024
NUB = 4096 // XB
UNITS = 200 * NUB
U_PER_W = UNITS // NUM_WORKERS


def _sc_xpose(y0, y1):
    mesh = plsc.VectorSubcoreMesh(core_axis_name="c", subcore_axis_name="s")

    @functools.partial(
        pl.kernel,
        mesh=mesh,
        out_type=jax.ShapeDtypeStruct((200, D, 4096), jnp.float32),
        compiler_params=pltpu.CompilerParams(
            use_tc_tiling_on_sc=False, needs_layout_passes=False),
        scratch_types=[
            pltpu.VMEM((XB, D), jnp.float32),
            pltpu.VMEM((D, XB), jnp.float32),
        ],
    )
    def xpose_kernel(y0_hbm, y1_hbm, out_hbm, in_v, out_v):
        wid = lax.axis_index("s") * NUM_CORES + lax.axis_index("c")
        lane = lax.iota(jnp.int32, 16)

        def unit(u, carry):
            g = wid * U_PER_W + u
            s = g // NUB
            b0 = (g % NUB) * XB

            @pl.when(s < S_PER)
            def _():
                pltpu.sync_copy(y0_hbm.at[pl.ds(b0, XB), s], in_v)

            @pl.when(s >= S_PER)
            def _():
                pltpu.sync_copy(y1_hbm.at[pl.ds(b0, XB), s - S_PER], in_v)

            def chunk(k, c2):
                row = k * 16 + lane
                for d in range(D):
                    vals = plsc.load_gather(in_v, [row, jnp.full((16,), d, jnp.int32)])
                    out_v[d, pl.ds(k * 16, 16)] = vals
                return c2

            lax.fori_loop(0, XB // 16, chunk, 0)
            pltpu.sync_copy(out_v, out_hbm.at[s, :, pl.ds(b0, XB)])
            return carry

        lax.fori_loop(0, U_PER_W, unit, 0)

    return xpose_kernel(y0, y1)


def kernel(input_ids, W):
    table = _pack_table(W).reshape(VP, D)
    halves = []
    for h in range(N_SPLIT):
        ih = input_ids[:, h * S_PER:(h + 1) * S_PER]
        idx = ih.reshape(-1).astype(jnp.int32)
        j = (idx % V4P) * 4 + idx // V4P
        rows = _sc_gather(j, table, NB_H)
        out_h = _tc_project(rows.reshape(NB_H * D // 128, 128), NB_H * D // 128)
        halves.append(out_h.reshape(4096, S_PER, D))
    out_phys = _sc_xpose(halves[0], halves[1])
    return out_phys.transpose(2, 0, 1)


# speedup vs baseline: 1.4609x; 1.4609x over previous
import functools

import jax
import jax.numpy as jnp
import numpy as np
from jax import lax
from jax.experimental import pallas as pl
from jax.experimental.pallas import tpu as pltpu
from jax.experimental.pallas import tpu_sc as plsc

EPS = 1e-6

NUM_CORES = 2
NUM_SUBCORES = 16
NUM_WORKERS = NUM_CORES * NUM_SUBCORES

B_TOTAL = 4096 * 200
D = 32
CHUNK = 3200


V = 1000000
BQ = 2048
NQ = 123
V4P = NQ * BQ
VP = 4 * V4P
_LAST_CBLK = (V - 1) // BQ


def _xp_body(a_ref, b_ref, c_ref, d_ref, out_ref):
    out_ref[:, 0:32] = a_ref[...].T
    out_ref[:, 32:64] = b_ref[...].T
    out_ref[:, 64:96] = c_ref[...].T
    out_ref[:, 96:128] = d_ref[...].T


def _pack_table(W):
    WT = W.T
    return pl.pallas_call(
        _xp_body,
        grid=(NQ,),
        in_specs=[
            pl.BlockSpec(
                (32, BQ),
                lambda i, r=r: (0, jnp.minimum(r * NQ + i, _LAST_CBLK)))
            for r in range(4)
        ],
        out_specs=pl.BlockSpec((BQ, 128), lambda i: (i, 0)),
        out_shape=jax.ShapeDtypeStruct((V4P, 128), jnp.float32),
    )(WT, WT, WT, WT)


def _sc_gather(idx, table, nb):
    b_per_w = nb // NUM_WORKERS
    n_chunks = b_per_w // CHUNK
    assert b_per_w % CHUNK == 0
    mesh = plsc.VectorSubcoreMesh(core_axis_name="c", subcore_axis_name="s")

    @functools.partial(
        pl.kernel,
        mesh=mesh,
        out_type=jax.ShapeDtypeStruct((nb, D), jnp.float32),
        compiler_params=pltpu.CompilerParams(use_tc_tiling_on_sc=False),
        scratch_types=[
            pltpu.VMEM((CHUNK,), jnp.int32),
            pltpu.VMEM((CHUNK, D), jnp.float32),
            pltpu.SemaphoreType.DMA,
        ],
    )
    def gather_kernel(idx_hbm, table_hbm, out_hbm, idx_v, rows_v, sem):
        wid = lax.axis_index("s") * NUM_CORES + lax.axis_index("c")
        base = wid * b_per_w

        def body(i, carry):
            off = base + i * CHUNK
            pltpu.sync_copy(idx_hbm.at[pl.ds(off, CHUNK)], idx_v)
            pltpu.async_copy(table_hbm.at[idx_v], rows_v, sem).wait()
            pltpu.sync_copy(rows_v, out_hbm.at[pl.ds(off, CHUNK)])
            return carry

        lax.fori_loop(0, n_chunks, body, 0)

    return gather_kernel(idx, table)


PACK = 128 // D
N_DENSE = B_TOTAL // PACK
BLK = 6400


def _tc_body(rows_ref, ones_ref, out_ref):
    x = rows_ref[...]
    s = jax.lax.dot_general(
        x * x, ones_ref[...], (((1,), (0,)), ((), ())),
        precision=jax.lax.Precision.HIGHEST,
        preferred_element_type=jnp.float32,
    )
    x0 = jnp.sqrt(1.0 + s)
    z = jnp.maximum(x0, 1.0 + EPS)
    m = jnp.maximum(x0 * x0 - 1.0, EPS)
    r = jax.lax.rsqrt(m)
    alpha = jnp.log(z + m * r)
    out_ref[...] = (alpha * r) * x


def _tc_project(rows_dense, nd):
    ones_bd = jnp.asarray(
        np.kron(np.eye(PACK, dtype=np.float32), np.ones((D, D), np.float32)))
    grid = (nd // BLK,)
    return pl.pallas_call(
        _tc_body,
        grid=grid,
        in_specs=[
            pl.BlockSpec((BLK, 128), lambda i: (i, 0)),
            pl.BlockSpec((128, 128), lambda i: (0, 0)),
        ],
        out_specs=pl.BlockSpec((BLK, 128), lambda i: (i, 0)),
        out_shape=jax.ShapeDtypeStruct((nd, 128), jnp.float32),
    )(rows_dense, ones_bd)


N_SPLIT = 4
S_PER = 200 // N_SPLIT


def kernel(input_ids, W):
    table = _pack_table(W).reshape(VP, D)
    outs = []
    for h in range(N_SPLIT):
        ih = input_ids[:, h * S_PER:(h + 1) * S_PER]
        idx = ih.reshape(-1).astype(jnp.int32)
        j = (idx % V4P) * 4 + idx // V4P
        nb = 4096 * S_PER
        rows = _sc_gather(j, table, nb)
        out_h = _tc_project(rows.reshape(nb * D // 128, 128), nb * D // 128)
        outs.append(out_h.reshape(4096, S_PER, D))
    return jnp.concatenate(outs, axis=1)


# speedup vs baseline: 1.5008x; 1.0274x over previous
import functools

import jax
import jax.numpy as jnp
import numpy as np
from jax import lax
from jax.experimental import pallas as pl
from jax.experimental.pallas import tpu as pltpu
from jax.experimental.pallas import tpu_sc as plsc

EPS = 1e-6

NUM_CORES = 2
NUM_SUBCORES = 16
NUM_WORKERS = NUM_CORES * NUM_SUBCORES

B_TOTAL = 4096 * 200
D = 32
CHUNK = 3200


V = 1000000
BQ = 2048
NQ = 123
V4P = NQ * BQ
VP = 4 * V4P
_LAST_CBLK = (V - 1) // BQ


def _xp_body(a_ref, b_ref, c_ref, d_ref, out_ref):
    out_ref[:, 0:32] = a_ref[...].T
    out_ref[:, 32:64] = b_ref[...].T
    out_ref[:, 64:96] = c_ref[...].T
    out_ref[:, 96:128] = d_ref[...].T


def _pack_table(W):
    WT = W.T
    return pl.pallas_call(
        _xp_body,
        grid=(NQ,),
        in_specs=[
            pl.BlockSpec(
                (32, BQ),
                lambda i, r=r: (0, jnp.minimum(r * NQ + i, _LAST_CBLK)))
            for r in range(4)
        ],
        out_specs=pl.BlockSpec((BQ, 128), lambda i: (i, 0)),
        out_shape=jax.ShapeDtypeStruct((V4P, 128), jnp.float32),
    )(WT, WT, WT, WT)


def _sc_gather(idx, table, nb):
    b_per_w = nb // NUM_WORKERS
    n_chunks = b_per_w // CHUNK
    assert b_per_w % CHUNK == 0
    mesh = plsc.VectorSubcoreMesh(core_axis_name="c", subcore_axis_name="s")

    @functools.partial(
        pl.kernel,
        mesh=mesh,
        out_type=jax.ShapeDtypeStruct((nb, D), jnp.float32),
        compiler_params=pltpu.CompilerParams(use_tc_tiling_on_sc=False),
        scratch_types=[
            pltpu.VMEM((CHUNK,), jnp.int32),
            pltpu.VMEM((CHUNK, D), jnp.float32),
            pltpu.SemaphoreType.DMA,
        ],
    )
    def gather_kernel(idx_hbm, table_hbm, out_hbm, idx_v, rows_v, sem):
        wid = lax.axis_index("s") * NUM_CORES + lax.axis_index("c")
        base = wid * b_per_w

        def body(i, carry):
            off = base + i * CHUNK
            pltpu.sync_copy(idx_hbm.at[pl.ds(off, CHUNK)], idx_v)
            pltpu.async_copy(table_hbm.at[idx_v], rows_v, sem).wait()
            pltpu.sync_copy(rows_v, out_hbm.at[pl.ds(off, CHUNK)])
            return carry

        lax.fori_loop(0, n_chunks, body, 0)

    return gather_kernel(idx, table)


PACK = 128 // D
N_DENSE = B_TOTAL // PACK
BLK = 6400


def _tc_body(rows_ref, ones_ref, out_ref):
    x = rows_ref[...]
    s = jax.lax.dot_general(
        x * x, ones_ref[...], (((1,), (0,)), ((), ())),
        precision=jax.lax.Precision.HIGHEST,
        preferred_element_type=jnp.float32,
    )
    x0 = jnp.sqrt(1.0 + s)
    z = jnp.maximum(x0, 1.0 + EPS)
    m = jnp.maximum(x0 * x0 - 1.0, EPS)
    r = jax.lax.rsqrt(m)
    alpha = jnp.log(z + m * r)
    out_ref[...] = (alpha * r) * x


def _tc_project(rows_dense, nd):
    ones_bd = jnp.asarray(
        np.kron(np.eye(PACK, dtype=np.float32), np.ones((D, D), np.float32)))
    grid = (nd // BLK,)
    return pl.pallas_call(
        _tc_body,
        grid=grid,
        in_specs=[
            pl.BlockSpec((BLK, 128), lambda i: (i, 0)),
            pl.BlockSpec((128, 128), lambda i: (0, 0)),
        ],
        out_specs=pl.BlockSpec((BLK, 128), lambda i: (i, 0)),
        out_shape=jax.ShapeDtypeStruct((nd, 128), jnp.float32),
    )(rows_dense, ones_bd)


N_SPLIT = 4
S_PER = 200 // N_SPLIT


def kernel(input_ids, W):
    table = _pack_table(W).reshape(VP, D)
    outs = []
    for h in range(N_SPLIT):
        ih = input_ids[:, h * S_PER:(h + 1) * S_PER]
        idx = ih.reshape(-1).astype(jnp.uint32)
        j = ((idx % V4P) * 4 + idx // V4P).astype(jnp.int32)
        nb = 4096 * S_PER
        rows = _sc_gather(j, table, nb)
        out_h = _tc_project(rows.reshape(nb * D // 128, 128), nb * D // 128)
        outs.append(out_h.reshape(4096, S_PER, D))
    return jnp.concatenate(outs, axis=1)


# speedup vs baseline: 1.5575x; 1.0378x over previous
import functools

import jax
import jax.numpy as jnp
import numpy as np
from jax import lax
from jax.experimental import pallas as pl
from jax.experimental.pallas import tpu as pltpu
from jax.experimental.pallas import tpu_sc as plsc

EPS = 1e-6

NUM_CORES = 2
NUM_SUBCORES = 16
NUM_WORKERS = NUM_CORES * NUM_SUBCORES

B_TOTAL = 4096 * 200
D = 32
CHUNK = 3200


V = 1000000
BQ = 2048
NQ = 123
V4P = NQ * BQ
VP = 4 * V4P
_LAST_CBLK = (V - 1) // BQ


def _xp_body(a_ref, b_ref, c_ref, d_ref, out_ref):
    out_ref[:, 0:32] = a_ref[...].T
    out_ref[:, 32:64] = b_ref[...].T
    out_ref[:, 64:96] = c_ref[...].T
    out_ref[:, 96:128] = d_ref[...].T


def _pack_table(W):
    WT = W.T
    return pl.pallas_call(
        _xp_body,
        grid=(NQ,),
        in_specs=[
            pl.BlockSpec(
                (32, BQ),
                lambda i, r=r: (0, jnp.minimum(r * NQ + i, _LAST_CBLK)))
            for r in range(4)
        ],
        out_specs=pl.BlockSpec((BQ, 128), lambda i: (i, 0)),
        out_shape=jax.ShapeDtypeStruct((V4P, 128), jnp.float32),
    )(WT, WT, WT, WT)


def _sc_gather(idx, table, nb):
    b_per_w = nb // NUM_WORKERS
    n_chunks = b_per_w // CHUNK
    assert b_per_w % CHUNK == 0
    mesh = plsc.VectorSubcoreMesh(core_axis_name="c", subcore_axis_name="s")

    @functools.partial(
        pl.kernel,
        mesh=mesh,
        out_type=jax.ShapeDtypeStruct((nb, D), jnp.float32),
        compiler_params=pltpu.CompilerParams(use_tc_tiling_on_sc=False),
        scratch_types=[
            pltpu.VMEM((CHUNK,), jnp.int32),
            pltpu.VMEM((CHUNK, D), jnp.float32),
            pltpu.SemaphoreType.DMA,
        ],
    )
    def gather_kernel(idx_hbm, table_hbm, out_hbm, idx_v, rows_v, sem):
        wid = lax.axis_index("s") * NUM_CORES + lax.axis_index("c")
        base = wid * b_per_w

        def body(i, carry):
            off = base + i * CHUNK
            pltpu.sync_copy(idx_hbm.at[pl.ds(off, CHUNK)], idx_v)
            pltpu.async_copy(table_hbm.at[idx_v], rows_v, sem).wait()
            pltpu.sync_copy(rows_v, out_hbm.at[pl.ds(off, CHUNK)])
            return carry

        lax.fori_loop(0, n_chunks, body, 0)

    return gather_kernel(idx, table)


PACK = 128 // D
N_DENSE = B_TOTAL // PACK
BLK = 6400


def _tc_body(rows_ref, ones_ref, out_ref):
    x = rows_ref[...]
    s = jax.lax.dot_general(
        x * x, ones_ref[...], (((1,), (0,)), ((), ())),
        preferred_element_type=jnp.float32,
    )
    x0 = jnp.sqrt(1.0 + s)
    z = jnp.maximum(x0, 1.0 + EPS)
    m = jnp.maximum(x0 * x0 - 1.0, EPS)
    r = jax.lax.rsqrt(m)
    alpha = jnp.log(z + m * r)
    out_ref[...] = (alpha * r) * x


def _tc_project(rows_dense, nd):
    ones_bd = jnp.asarray(
        np.kron(np.eye(PACK, dtype=np.float32), np.ones((D, D), np.float32)))
    grid = (nd // BLK,)
    return pl.pallas_call(
        _tc_body,
        grid=grid,
        in_specs=[
            pl.BlockSpec((BLK, 128), lambda i: (i, 0)),
            pl.BlockSpec((128, 128), lambda i: (0, 0)),
        ],
        out_specs=pl.BlockSpec((BLK, 128), lambda i: (i, 0)),
        out_shape=jax.ShapeDtypeStruct((nd, 128), jnp.float32),
    )(rows_dense, ones_bd)


N_SPLIT = 4
S_PER = 200 // N_SPLIT


def kernel(input_ids, W):
    table = _pack_table(W).reshape(VP, D)
    outs = []
    for h in range(N_SPLIT):
        ih = input_ids[:, h * S_PER:(h + 1) * S_PER]
        idx = ih.reshape(-1).astype(jnp.uint32)
        j = ((idx % V4P) * 4 + idx // V4P).astype(jnp.int32)
        nb = 4096 * S_PER
        rows = _sc_gather(j, table, nb)
        out_h = _tc_project(rows.reshape(nb * D // 128, 128), nb * D // 128)
        outs.append(out_h.reshape(4096, S_PER, D))
    return jnp.concatenate(outs, axis=1)


# speedup vs baseline: 1.5723x; 1.0095x over previous
import functools

import jax
import jax.numpy as jnp
import numpy as np
from jax import lax
from jax.experimental import pallas as pl
from jax.experimental.pallas import tpu as pltpu
from jax.experimental.pallas import tpu_sc as plsc

EPS = 1e-6

NUM_CORES = 2
NUM_SUBCORES = 16
NUM_WORKERS = NUM_CORES * NUM_SUBCORES

B_TOTAL = 4096 * 200
D = 32
CHUNK = 3200


V = 1000000
BQ = 4096
NQ = 62
V4P = NQ * BQ
VP = 4 * V4P
_LAST_CBLK = (V - 1) // BQ


def _xp_body(a_ref, b_ref, c_ref, d_ref, out_ref):
    out_ref[:, 0:32] = a_ref[...].T
    out_ref[:, 32:64] = b_ref[...].T
    out_ref[:, 64:96] = c_ref[...].T
    out_ref[:, 96:128] = d_ref[...].T


def _pack_table(W):
    WT = W.T
    return pl.pallas_call(
        _xp_body,
        grid=(NQ,),
        in_specs=[
            pl.BlockSpec(
                (32, BQ),
                lambda i, r=r: (0, jnp.minimum(r * NQ + i, _LAST_CBLK)))
            for r in range(4)
        ],
        out_specs=pl.BlockSpec((BQ, 128), lambda i: (i, 0)),
        out_shape=jax.ShapeDtypeStruct((V4P, 128), jnp.float32),
    )(WT, WT, WT, WT)


def _sc_gather(idx, table, nb):
    b_per_w = nb // NUM_WORKERS
    n_chunks = b_per_w // CHUNK
    assert b_per_w % CHUNK == 0
    mesh = plsc.VectorSubcoreMesh(core_axis_name="c", subcore_axis_name="s")

    @functools.partial(
        pl.kernel,
        mesh=mesh,
        out_type=jax.ShapeDtypeStruct((nb, D), jnp.float32),
        compiler_params=pltpu.CompilerParams(use_tc_tiling_on_sc=False),
        scratch_types=[
            pltpu.VMEM((CHUNK,), jnp.int32),
            pltpu.VMEM((CHUNK, D), jnp.float32),
            pltpu.SemaphoreType.DMA,
        ],
    )
    def gather_kernel(idx_hbm, table_hbm, out_hbm, idx_v, rows_v, sem):
        wid = lax.axis_index("s") * NUM_CORES + lax.axis_index("c")
        base = wid * b_per_w

        def body(i, carry):
            off = base + i * CHUNK
            pltpu.sync_copy(idx_hbm.at[pl.ds(off, CHUNK)], idx_v)
            pltpu.async_copy(table_hbm.at[idx_v], rows_v, sem).wait()
            pltpu.sync_copy(rows_v, out_hbm.at[pl.ds(off, CHUNK)])
            return carry

        lax.fori_loop(0, n_chunks, body, 0)

    return gather_kernel(idx, table)


PACK = 128 // D
N_DENSE = B_TOTAL // PACK
BLK = 6400


def _tc_body(rows_ref, ones_ref, out_ref):
    x = rows_ref[...]
    s = jax.lax.dot_general(
        x * x, ones_ref[...], (((1,), (0,)), ((), ())),
        preferred_element_type=jnp.float32,
    )
    x0 = jnp.sqrt(1.0 + s)
    z = jnp.maximum(x0, 1.0 + EPS)
    m = jnp.maximum(x0 * x0 - 1.0, EPS)
    r = jax.lax.rsqrt(m)
    alpha = jnp.log(z + m * r)
    out_ref[...] = (alpha * r) * x


def _tc_project(rows_dense, nd):
    ones_bd = jnp.asarray(
        np.kron(np.eye(PACK, dtype=np.float32), np.ones((D, D), np.float32)))
    grid = (nd // BLK,)
    return pl.pallas_call(
        _tc_body,
        grid=grid,
        in_specs=[
            pl.BlockSpec((BLK, 128), lambda i: (i, 0)),
            pl.BlockSpec((128, 128), lambda i: (0, 0)),
        ],
        out_specs=pl.BlockSpec((BLK, 128), lambda i: (i, 0)),
        out_shape=jax.ShapeDtypeStruct((nd, 128), jnp.float32),
    )(rows_dense, ones_bd)


N_SPLIT = 4
S_PER = 200 // N_SPLIT


def kernel(input_ids, W):
    table = _pack_table(W).reshape(VP, D)
    outs = []
    for h in range(N_SPLIT):
        ih = input_ids[:, h * S_PER:(h + 1) * S_PER]
        idx = ih.reshape(-1).astype(jnp.uint32)
        j = ((idx % V4P) * 4 + idx // V4P).astype(jnp.int32)
        nb = 4096 * S_PER
        rows = _sc_gather(j, table, nb)
        out_h = _tc_project(rows.reshape(nb * D // 128, 128), nb * D // 128)
        outs.append(out_h.reshape(4096, S_PER, D))
    return jnp.concatenate(outs, axis=1)


# speedup vs baseline: 1.5768x; 1.0029x over previous
import functools

import jax
import jax.numpy as jnp
import numpy as np
from jax import lax
from jax.experimental import pallas as pl
from jax.experimental.pallas import tpu as pltpu
from jax.experimental.pallas import tpu_sc as plsc

EPS = 1e-6

NUM_CORES = 2
NUM_SUBCORES = 16
NUM_WORKERS = NUM_CORES * NUM_SUBCORES

B_TOTAL = 4096 * 200
D = 32
CHUNK = 3200


V = 1000000
BQ = 8192
NQ = 31
V4P = NQ * BQ
VP = 4 * V4P
_LAST_CBLK = (V - 1) // BQ


def _xp_body(a_ref, b_ref, c_ref, d_ref, out_ref):
    out_ref[:, 0:32] = a_ref[...].T
    out_ref[:, 32:64] = b_ref[...].T
    out_ref[:, 64:96] = c_ref[...].T
    out_ref[:, 96:128] = d_ref[...].T


def _pack_table(W):
    WT = W.T
    return pl.pallas_call(
        _xp_body,
        grid=(NQ,),
        in_specs=[
            pl.BlockSpec(
                (32, BQ),
                lambda i, r=r: (0, jnp.minimum(r * NQ + i, _LAST_CBLK)))
            for r in range(4)
        ],
        out_specs=pl.BlockSpec((BQ, 128), lambda i: (i, 0)),
        out_shape=jax.ShapeDtypeStruct((V4P, 128), jnp.float32),
    )(WT, WT, WT, WT)


def _sc_gather(idx, table, nb):
    b_per_w = nb // NUM_WORKERS
    n_chunks = b_per_w // CHUNK
    assert b_per_w % CHUNK == 0
    mesh = plsc.VectorSubcoreMesh(core_axis_name="c", subcore_axis_name="s")

    @functools.partial(
        pl.kernel,
        mesh=mesh,
        out_type=jax.ShapeDtypeStruct((nb, D), jnp.float32),
        compiler_params=pltpu.CompilerParams(use_tc_tiling_on_sc=False),
        scratch_types=[
            pltpu.VMEM((CHUNK,), jnp.int32),
            pltpu.VMEM((CHUNK, D), jnp.float32),
            pltpu.SemaphoreType.DMA,
        ],
    )
    def gather_kernel(idx_hbm, table_hbm, out_hbm, idx_v, rows_v, sem):
        wid = lax.axis_index("s") * NUM_CORES + lax.axis_index("c")
        base = wid * b_per_w

        def body(i, carry):
            off = base + i * CHUNK
            pltpu.sync_copy(idx_hbm.at[pl.ds(off, CHUNK)], idx_v)
            pltpu.async_copy(table_hbm.at[idx_v], rows_v, sem).wait()
            pltpu.sync_copy(rows_v, out_hbm.at[pl.ds(off, CHUNK)])
            return carry

        lax.fori_loop(0, n_chunks, body, 0)

    return gather_kernel(idx, table)


PACK = 128 // D
N_DENSE = B_TOTAL // PACK
BLK = 6400


def _tc_body(rows_ref, ones_ref, out_ref):
    x = rows_ref[...]
    s = jax.lax.dot_general(
        x * x, ones_ref[...], (((1,), (0,)), ((), ())),
        preferred_element_type=jnp.float32,
    )
    x0 = jnp.sqrt(1.0 + s)
    z = jnp.maximum(x0, 1.0 + EPS)
    m = jnp.maximum(x0 * x0 - 1.0, EPS)
    r = jax.lax.rsqrt(m)
    alpha = jnp.log(z + m * r)
    out_ref[...] = (alpha * r) * x


def _tc_project(rows_dense, nd):
    ones_bd = jnp.asarray(
        np.kron(np.eye(PACK, dtype=np.float32), np.ones((D, D), np.float32)))
    grid = (nd // BLK,)
    return pl.pallas_call(
        _tc_body,
        grid=grid,
        in_specs=[
            pl.BlockSpec((BLK, 128), lambda i: (i, 0)),
            pl.BlockSpec((128, 128), lambda i: (0, 0)),
        ],
        out_specs=pl.BlockSpec((BLK, 128), lambda i: (i, 0)),
        out_shape=jax.ShapeDtypeStruct((nd, 128), jnp.float32),
    )(rows_dense, ones_bd)


N_SPLIT = 4
S_PER = 200 // N_SPLIT


def kernel(input_ids, W):
    table = _pack_table(W).reshape(VP, D)
    outs = []
    for h in range(N_SPLIT):
        ih = input_ids[:, h * S_PER:(h + 1) * S_PER]
        idx = ih.reshape(-1).astype(jnp.uint32)
        j = ((idx % V4P) * 4 + idx // V4P).astype(jnp.int32)
        nb = 4096 * S_PER
        rows = _sc_gather(j, table, nb)
        out_h = _tc_project(rows.reshape(nb * D // 128, 128), nb * D // 128)
        outs.append(out_h.reshape(4096, S_PER, D))
    return jnp.concatenate(outs, axis=1)
